# baseline (speedup 1.0000x reference)
import functools

import jax
import jax.numpy as jnp
from jax import lax
from jax.experimental import pallas as pl
from jax.experimental.pallas import tpu as pltpu
from jax.experimental.pallas import tpu_sc as plsc

B = 8
S1 = 2047
TOTAL = 1391
INST_START = 852
WIN_LO = 768
WIN_HI = 1024
WOFF = INST_START - WIN_LO
TS = 256
NSB = 8
SP = 2048
CH = 512
SONG_PAD = 2056
DUMMY_BIT = 255

NEG = -1e9


def _sc_mask_body(song_ref, ct_ref, p_ref, ex_ref,
                  song_v, ct_v, scat_v, ftab_v, words_v, tmp_v, acc_v,
                  stage_v):
    cid = lax.axis_index("c")
    sid = lax.axis_index("s")
    wid = cid * 16 + sid
    b = wid // 4
    s = wid % 4
    i0 = s * CH

    pltpu.sync_copy(song_ref.at[b, pl.ds(i0, CH + 8), :], song_v)
    pltpu.sync_copy(ct_ref.at[b, pl.ds(i0, CH)], ct_v)

    iota = lax.iota(jnp.int32, 16)

    def ext(g, carry):
        t = jnp.minimum(iota + g * 16, CH)
        flags = plsc.load_gather(song_v, [t, jnp.zeros_like(iota)])
        vals = plsc.load_gather(song_v, [t, jnp.full_like(iota, 6)])
        scat = jnp.where(flags == 1, vals + WOFF, DUMMY_BIT)
        scat_v[pl.ds(g * 16, 16)] = scat
        return carry
    lax.fori_loop(0, 33, ext, 0)

    def finit(j, carry):
        ftab_v[pl.ds(j * 16, 16)] = jnp.ones((16,), jnp.float32)
        return carry
    lax.fori_loop(0, 16, finit, 0)

    def pha(g, carry):
        idx = scat_v[pl.ds(g * 16, 16)]
        plsc.store_scatter(ftab_v, [idx], jnp.zeros((16,), jnp.float32))
        return carry
    lax.fori_loop(0, 32, pha, 0)

    def conv(wj, wvec):
        g0 = ftab_v[pl.ds(wj * 32, 16)]
        g1 = ftab_v[pl.ds(wj * 32 + 16, 16)]
        lo = jnp.sum(jnp.where(g0 == 0.0, jnp.int32(1) << iota, 0))
        hi = jnp.sum(jnp.where(g1 == 0.0, jnp.int32(1) << iota, 0))
        word = lo | (hi << 16)
        return wvec | jnp.where(iota == wj, word, 0)
    words = lax.fori_loop(0, 8, conv, jnp.zeros((16,), jnp.int32))
    words_v[...] = words

    pltpu.sync_copy(words_v, ex_ref.at[wid])
    plsc.subcore_barrier()
    acc_v[...] = jnp.zeros((16,), jnp.int32)
    for k in range(3):
        @pl.when(k < s)
        def _(k=k):
            pltpu.sync_copy(ex_ref.at[b * 4 + k], tmp_v)
            acc_v[...] = acc_v[...] | tmp_v[...]

    tok0 = scat_v[pl.ds(0, 16)][0]
    tbl0 = acc_v[...] | jnp.where(iota == (tok0 >> 5), 1 << (tok0 & 31), 0)

    def emit_grp(g, tbl):
        tok16 = scat_v[pl.ds(g * 16 + 1, 16)]
        ct16 = ct_v[pl.ds(g * 16, 16)]
        for j in range(16):
            tok = tok16[j]
            tbl = tbl | jnp.where(iota == (tok >> 5), 1 << (tok & 31), 0)
            cvec = jnp.broadcast_to(ct16[j], (16,)) == 1
            out = jnp.where(cvec, tbl, jnp.zeros((16,), jnp.int32))
            stage_v[pl.ds((g * 16 + j) * 16, 16)] = out
        return tbl
    lax.fori_loop(0, CH // 16, emit_grp, tbl0)

    pltpu.sync_copy(stage_v, p_ref.at[b, pl.ds(i0 * 16, CH * 16)])


def _build_sc(interpret=False):
    mesh = plsc.VectorSubcoreMesh(core_axis_name="c", subcore_axis_name="s",
                                  num_cores=2, num_subcores=16)
    return pl.kernel(
        _sc_mask_body,
        out_type=(jax.ShapeDtypeStruct((B, SP * 16), jnp.int32),
                  jax.ShapeDtypeStruct((32, 16), jnp.int32)),
        mesh=mesh,
        scratch_types=[
            pltpu.VMEM((CH + 8, 11), jnp.int32),
            pltpu.VMEM((CH,), jnp.int32),
            pltpu.VMEM((528,), jnp.int32),
            pltpu.VMEM((256,), jnp.float32),
            pltpu.VMEM((16,), jnp.int32),
            pltpu.VMEM((16,), jnp.int32),
            pltpu.VMEM((16,), jnp.int32),
            pltpu.VMEM((CH * 16,), jnp.int32),
        ],
        compiler_params=pltpu.CompilerParams(needs_layout_passes=False),
        interpret=interpret,
    )


def _tc_apply_body(p_ref, x_ref, o_ref):
    words = p_ref[0]
    lanes = jax.lax.broadcasted_iota(jnp.int32, (TS, 256), 1)
    shamt = lanes & 31
    grp = lanes >> 5
    acc = jnp.zeros((TS, 256), jnp.int32)
    for wj in range(2, 7):
        wcol = words[:, wj:wj + 1]
        bits = (wcol >> shamt) & 1
        acc = acc | jnp.where(grp == wj, bits, 0)

    o_ref[0, :, :WIN_LO] = x_ref[0, :, :WIN_LO]
    o_ref[0, :, WIN_LO:WIN_HI] = jnp.where(
        acc == 1, NEG, x_ref[0, :, WIN_LO:WIN_HI])
    o_ref[0, :, WIN_HI:] = x_ref[0, :, WIN_HI:]


def _build_tc(interpret=False):
    return pl.pallas_call(
        _tc_apply_body,
        grid=(B, NSB),
        in_specs=[
            pl.BlockSpec((1, TS, 16), lambda b, s: (b, s, 0)),
            pl.BlockSpec((1, TS, TOTAL), lambda b, s: (b, s, 0)),
        ],
        out_specs=pl.BlockSpec((1, TS, TOTAL), lambda b, s: (b, s, 0)),
        out_shape=jax.ShapeDtypeStruct((B, S1, TOTAL), jnp.float32),
        interpret=interpret,
    )


def kernel(chosen_types, song_tokens, seq_scores):
    song = song_tokens.astype(jnp.int32)
    song = jnp.pad(song, ((0, 0), (0, SONG_PAD - S1), (0, 0)))
    ct = jnp.pad(chosen_types.astype(jnp.int32), ((0, 0), (0, SP - S1)))
    p, _ = _build_sc()(song, ct)
    p3 = p.reshape(B, SP, 16)
    return _build_tc()(p3, seq_scores)


# speedup vs baseline: 1.0534x; 1.0534x over previous
import functools

import jax
import jax.numpy as jnp
from jax import lax
from jax.experimental import pallas as pl
from jax.experimental.pallas import tpu as pltpu
from jax.experimental.pallas import tpu_sc as plsc

B = 8
S1 = 2047
TOTAL = 1391
INST_START = 852
WIN_LO = 768
WIN_HI = 1024
WOFF = INST_START - WIN_LO
TS = 512
NSB = 4
SP = 2048
CH = 512
SONG_PAD = 2056
DUMMY_BIT = 255

NEG = -1e9


def _sc_mask_body(song_ref, ct_ref, p_ref, ex_ref,
                  song_v, ct_v, scat_v, ftab_v, words_v, tmp_v, acc_v,
                  stage_v):
    cid = lax.axis_index("c")
    sid = lax.axis_index("s")
    wid = cid * 16 + sid
    b = wid // 4
    s = wid % 4
    i0 = s * CH

    pltpu.sync_copy(song_ref.at[b, pl.ds(i0, CH + 8), :], song_v)
    pltpu.sync_copy(ct_ref.at[b, pl.ds(i0, CH)], ct_v)

    iota = lax.iota(jnp.int32, 16)

    def ext(g, carry):
        t = jnp.minimum(iota + g * 16, CH)
        flags = plsc.load_gather(song_v, [t, jnp.zeros_like(iota)])
        vals = plsc.load_gather(song_v, [t, jnp.full_like(iota, 6)])
        scat = jnp.where(flags == 1, vals + WOFF, DUMMY_BIT)
        scat_v[pl.ds(g * 16, 16)] = scat
        return carry
    lax.fori_loop(0, 33, ext, 0)

    def finit(j, carry):
        ftab_v[pl.ds(j * 16, 16)] = jnp.ones((16,), jnp.float32)
        return carry
    lax.fori_loop(0, 16, finit, 0)

    def pha(g, carry):
        idx = scat_v[pl.ds(g * 16, 16)]
        plsc.store_scatter(ftab_v, [idx], jnp.zeros((16,), jnp.float32))
        return carry
    lax.fori_loop(0, 32, pha, 0)

    def conv(wj, wvec):
        g0 = ftab_v[pl.ds(wj * 32, 16)]
        g1 = ftab_v[pl.ds(wj * 32 + 16, 16)]
        lo = jnp.sum(jnp.where(g0 == 0.0, jnp.int32(1) << iota, 0))
        hi = jnp.sum(jnp.where(g1 == 0.0, jnp.int32(1) << iota, 0))
        word = lo | (hi << 16)
        return wvec | jnp.where(iota == wj, word, 0)
    words = lax.fori_loop(0, 8, conv, jnp.zeros((16,), jnp.int32))
    words_v[...] = words

    pltpu.sync_copy(words_v, ex_ref.at[wid])
    plsc.subcore_barrier()
    acc_v[...] = jnp.zeros((16,), jnp.int32)
    for k in range(3):
        @pl.when(k < s)
        def _(k=k):
            pltpu.sync_copy(ex_ref.at[b * 4 + k], tmp_v)
            acc_v[...] = acc_v[...] | tmp_v[...]

    tok0 = scat_v[pl.ds(0, 16)][0]
    tbl0 = acc_v[...] | jnp.where(iota == (tok0 >> 5), 1 << (tok0 & 31), 0)

    def emit_grp(g, tbl):
        tok16 = scat_v[pl.ds(g * 16 + 1, 16)]
        ct16 = ct_v[pl.ds(g * 16, 16)]
        for j in range(16):
            tok = tok16[j]
            tbl = tbl | jnp.where(iota == (tok >> 5), 1 << (tok & 31), 0)
            cvec = jnp.broadcast_to(ct16[j], (16,)) == 1
            out = jnp.where(cvec, tbl, jnp.zeros((16,), jnp.int32))
            stage_v[pl.ds((g * 16 + j) * 16, 16)] = out
        return tbl
    lax.fori_loop(0, CH // 16, emit_grp, tbl0)

    pltpu.sync_copy(stage_v, p_ref.at[b, pl.ds(i0 * 16, CH * 16)])


def _build_sc(interpret=False):
    mesh = plsc.VectorSubcoreMesh(core_axis_name="c", subcore_axis_name="s",
                                  num_cores=2, num_subcores=16)
    return pl.kernel(
        _sc_mask_body,
        out_type=(jax.ShapeDtypeStruct((B, SP * 16), jnp.int32),
                  jax.ShapeDtypeStruct((32, 16), jnp.int32)),
        mesh=mesh,
        scratch_types=[
            pltpu.VMEM((CH + 8, 11), jnp.int32),
            pltpu.VMEM((CH,), jnp.int32),
            pltpu.VMEM((528,), jnp.int32),
            pltpu.VMEM((256,), jnp.float32),
            pltpu.VMEM((16,), jnp.int32),
            pltpu.VMEM((16,), jnp.int32),
            pltpu.VMEM((16,), jnp.int32),
            pltpu.VMEM((CH * 16,), jnp.int32),
        ],
        compiler_params=pltpu.CompilerParams(needs_layout_passes=False),
        interpret=interpret,
    )


def _tc_apply_body(p_ref, x_ref, o_ref):
    words = p_ref[0]
    lanes = jax.lax.broadcasted_iota(jnp.int32, (TS, 256), 1)
    shamt = lanes & 31
    grp = lanes >> 5
    acc = jnp.zeros((TS, 256), jnp.int32)
    for wj in range(2, 7):
        wcol = words[:, wj:wj + 1]
        bits = (wcol >> shamt) & 1
        acc = acc | jnp.where(grp == wj, bits, 0)

    o_ref[0, :, :WIN_LO] = x_ref[0, :, :WIN_LO]
    o_ref[0, :, WIN_LO:WIN_HI] = jnp.where(
        acc == 1, NEG, x_ref[0, :, WIN_LO:WIN_HI])
    o_ref[0, :, WIN_HI:] = x_ref[0, :, WIN_HI:]


def _build_tc(interpret=False):
    return pl.pallas_call(
        _tc_apply_body,
        grid=(B, NSB),
        in_specs=[
            pl.BlockSpec((1, TS, 16), lambda b, s: (b, s, 0)),
            pl.BlockSpec((1, TS, TOTAL), lambda b, s: (b, s, 0)),
        ],
        out_specs=pl.BlockSpec((1, TS, TOTAL), lambda b, s: (b, s, 0)),
        out_shape=jax.ShapeDtypeStruct((B, S1, TOTAL), jnp.float32),
        interpret=interpret,
    )


def kernel(chosen_types, song_tokens, seq_scores):
    song = song_tokens.astype(jnp.int32)
    song = jnp.pad(song, ((0, 0), (0, SONG_PAD - S1), (0, 0)))
    ct = jnp.pad(chosen_types.astype(jnp.int32), ((0, 0), (0, SP - S1)))
    p, _ = _build_sc()(song, ct)
    p3 = p.reshape(B, SP, 16)
    return _build_tc()(p3, seq_scores)


# speedup vs baseline: 1.0648x; 1.0109x over previous
import functools

import jax
import jax.numpy as jnp
from jax import lax
from jax.experimental import pallas as pl
from jax.experimental.pallas import tpu as pltpu
from jax.experimental.pallas import tpu_sc as plsc

B = 8
S1 = 2047
TOTAL = 1391
INST_START = 852
WIN_LO = 768
WIN_HI = 1024
WOFF = INST_START - WIN_LO
TS = 1024
NSB = 2
SP = 2048
CH = 512
SONG_PAD = 2056
DUMMY_BIT = 255

NEG = -1e9


def _sc_mask_body(song_ref, ct_ref, p_ref, ex_ref,
                  song_v, ct_v, scat_v, ftab_v, words_v, tmp_v, acc_v,
                  stage_v):
    cid = lax.axis_index("c")
    sid = lax.axis_index("s")
    wid = cid * 16 + sid
    b = wid // 4
    s = wid % 4
    i0 = s * CH

    pltpu.sync_copy(song_ref.at[b, pl.ds(i0, CH + 8), :], song_v)
    pltpu.sync_copy(ct_ref.at[b, pl.ds(i0, CH)], ct_v)

    iota = lax.iota(jnp.int32, 16)

    def ext(g, carry):
        t = jnp.minimum(iota + g * 16, CH)
        flags = plsc.load_gather(song_v, [t, jnp.zeros_like(iota)])
        vals = plsc.load_gather(song_v, [t, jnp.full_like(iota, 6)])
        scat = jnp.where(flags == 1, vals + WOFF, DUMMY_BIT)
        scat_v[pl.ds(g * 16, 16)] = scat
        return carry
    lax.fori_loop(0, 33, ext, 0)

    def finit(j, carry):
        ftab_v[pl.ds(j * 16, 16)] = jnp.ones((16,), jnp.float32)
        return carry
    lax.fori_loop(0, 16, finit, 0)

    def pha(g, carry):
        idx = scat_v[pl.ds(g * 16, 16)]
        plsc.store_scatter(ftab_v, [idx], jnp.zeros((16,), jnp.float32))
        return carry
    lax.fori_loop(0, 32, pha, 0)

    def conv(wj, wvec):
        g0 = ftab_v[pl.ds(wj * 32, 16)]
        g1 = ftab_v[pl.ds(wj * 32 + 16, 16)]
        lo = jnp.sum(jnp.where(g0 == 0.0, jnp.int32(1) << iota, 0))
        hi = jnp.sum(jnp.where(g1 == 0.0, jnp.int32(1) << iota, 0))
        word = lo | (hi << 16)
        return wvec | jnp.where(iota == wj, word, 0)
    words = lax.fori_loop(0, 8, conv, jnp.zeros((16,), jnp.int32))
    words_v[...] = words

    pltpu.sync_copy(words_v, ex_ref.at[wid])
    plsc.subcore_barrier()
    acc_v[...] = jnp.zeros((16,), jnp.int32)
    for k in range(3):
        @pl.when(k < s)
        def _(k=k):
            pltpu.sync_copy(ex_ref.at[b * 4 + k], tmp_v)
            acc_v[...] = acc_v[...] | tmp_v[...]

    tok0 = scat_v[pl.ds(0, 16)][0]
    tbl0 = acc_v[...] | jnp.where(iota == (tok0 >> 5), 1 << (tok0 & 31), 0)

    def emit_grp(g, tbl):
        tok16 = scat_v[pl.ds(g * 16 + 1, 16)]
        ct16 = ct_v[pl.ds(g * 16, 16)]
        for j in range(16):
            tok = tok16[j]
            tbl = tbl | jnp.where(iota == (tok >> 5), 1 << (tok & 31), 0)
            cvec = jnp.broadcast_to(ct16[j], (16,)) == 1
            out = jnp.where(cvec, tbl, jnp.zeros((16,), jnp.int32))
            stage_v[pl.ds((g * 16 + j) * 16, 16)] = out
        return tbl
    lax.fori_loop(0, CH // 16, emit_grp, tbl0)

    pltpu.sync_copy(stage_v, p_ref.at[b, pl.ds(i0 * 16, CH * 16)])


def _build_sc(interpret=False):
    mesh = plsc.VectorSubcoreMesh(core_axis_name="c", subcore_axis_name="s",
                                  num_cores=2, num_subcores=16)
    return pl.kernel(
        _sc_mask_body,
        out_type=(jax.ShapeDtypeStruct((B, SP * 16), jnp.int32),
                  jax.ShapeDtypeStruct((32, 16), jnp.int32)),
        mesh=mesh,
        scratch_types=[
            pltpu.VMEM((CH + 8, 11), jnp.int32),
            pltpu.VMEM((CH,), jnp.int32),
            pltpu.VMEM((528,), jnp.int32),
            pltpu.VMEM((256,), jnp.float32),
            pltpu.VMEM((16,), jnp.int32),
            pltpu.VMEM((16,), jnp.int32),
            pltpu.VMEM((16,), jnp.int32),
            pltpu.VMEM((CH * 16,), jnp.int32),
        ],
        compiler_params=pltpu.CompilerParams(needs_layout_passes=False),
        interpret=interpret,
    )


def _tc_apply_body(p_ref, x_ref, o_ref):
    words = p_ref[0]
    lanes = jax.lax.broadcasted_iota(jnp.int32, (TS, 256), 1)
    shamt = lanes & 31
    grp = lanes >> 5
    acc = jnp.zeros((TS, 256), jnp.int32)
    for wj in range(2, 7):
        wcol = words[:, wj:wj + 1]
        bits = (wcol >> shamt) & 1
        acc = acc | jnp.where(grp == wj, bits, 0)

    o_ref[0, :, :WIN_LO] = x_ref[0, :, :WIN_LO]
    o_ref[0, :, WIN_LO:WIN_HI] = jnp.where(
        acc == 1, NEG, x_ref[0, :, WIN_LO:WIN_HI])
    o_ref[0, :, WIN_HI:] = x_ref[0, :, WIN_HI:]


def _build_tc(interpret=False):
    return pl.pallas_call(
        _tc_apply_body,
        grid=(B, NSB),
        in_specs=[
            pl.BlockSpec((1, TS, 16), lambda b, s: (b, s, 0)),
            pl.BlockSpec((1, TS, TOTAL), lambda b, s: (b, s, 0)),
        ],
        out_specs=pl.BlockSpec((1, TS, TOTAL), lambda b, s: (b, s, 0)),
        out_shape=jax.ShapeDtypeStruct((B, S1, TOTAL), jnp.float32),
        interpret=interpret,
    )


def kernel(chosen_types, song_tokens, seq_scores):
    song = song_tokens.astype(jnp.int32)
    song = jnp.pad(song, ((0, 0), (0, SONG_PAD - S1), (0, 0)))
    ct = jnp.pad(chosen_types.astype(jnp.int32), ((0, 0), (0, SP - S1)))
    p, _ = _build_sc()(song, ct)
    p3 = p.reshape(B, SP, 16)
    return _build_tc()(p3, seq_scores)
